# baseline (speedup 1.0000x reference)
import functools

import jax
import jax.numpy as jnp
from jax.experimental import pallas as pl
from jax.experimental.pallas import tpu as pltpu

_TB = 512
_FB = 2048
_SPLIT = 1


def _fused_ffn_kernel(x_ref, w1_ref, w2_ref, b1_ref, b2_ref, g_ref, bb_ref,
                      c0_ref, out_ref, normed_ref, s_ref, *, nj):
    j = pl.program_id(1)

    @pl.when(j == 0)
    def _init():
        xb = x_ref[...]
        mu = jnp.mean(xb, axis=1, keepdims=True)
        var = jnp.mean(jnp.square(xb), axis=1, keepdims=True) - jnp.square(mu)
        normed = (xb - mu) * jax.lax.rsqrt(var + 1e-5)
        normed_ref[...] = (normed * g_ref[...] + bb_ref[...]).astype(
            jnp.bfloat16)
        s_ref[...] = jax.nn.sigmoid(
            jnp.sum(xb * c0_ref[...], axis=1, keepdims=True))
        out_ref[...] = jnp.zeros_like(out_ref)

    fbk = w1_ref.shape[0] // _SPLIT
    normed = normed_ref[...]
    acc = None
    for k in range(_SPLIT):
        h = jax.lax.dot_general(
            normed, w1_ref[k * fbk:(k + 1) * fbk, :],
            (((1,), (1,)), ((), ())), preferred_element_type=jnp.float32)
        h = jnp.maximum(h + b1_ref[:, k * fbk:(k + 1) * fbk],
                        0.0).astype(jnp.bfloat16)
        d = jax.lax.dot_general(
            h, w2_ref[:, k * fbk:(k + 1) * fbk],
            (((1,), (1,)), ((), ())), preferred_element_type=jnp.float32)
        acc = d if acc is None else acc + d

    out_ref[...] += acc

    @pl.when(j == nj - 1)
    def _finish():
        s = s_ref[...]
        out_ref[...] = x_ref[...] + s * (out_ref[...] + b2_ref[...])


def kernel(input_features, expert_centroids, ln_g, ln_b, ff1_w, ff1_b,
           ff2_w, ff2_b):
    B, S, D = input_features.shape
    FF = ff1_w.shape[0]
    N = B * S
    x = input_features.reshape(N, D)
    w1b = ff1_w.astype(jnp.bfloat16)
    w2b = ff2_w.astype(jnp.bfloat16)
    b1 = ff1_b.reshape(1, FF)
    b2 = ff2_b.reshape(1, D)
    g = ln_g.reshape(1, D)
    bb = ln_b.reshape(1, D)
    c0 = expert_centroids[0].reshape(1, D)
    ni, nj = N // _TB, FF // _FB

    out = pl.pallas_call(
        functools.partial(_fused_ffn_kernel, nj=nj),
        grid=(ni, nj),
        in_specs=[
            pl.BlockSpec((_TB, D), lambda i, j: (i, 0)),
            pl.BlockSpec((_FB, D), lambda i, j: (j, 0)),
            pl.BlockSpec((D, _FB), lambda i, j: (0, j)),
            pl.BlockSpec((1, _FB), lambda i, j: (0, j)),
            pl.BlockSpec((1, D), lambda i, j: (0, 0)),
            pl.BlockSpec((1, D), lambda i, j: (0, 0)),
            pl.BlockSpec((1, D), lambda i, j: (0, 0)),
            pl.BlockSpec((1, D), lambda i, j: (0, 0)),
        ],
        out_specs=pl.BlockSpec((_TB, D), lambda i, j: (i, 0)),
        out_shape=jax.ShapeDtypeStruct((N, D), jnp.float32),
        scratch_shapes=[
            pltpu.VMEM((_TB, D), jnp.bfloat16),
            pltpu.VMEM((_TB, 1), jnp.float32),
        ],
        compiler_params=pltpu.CompilerParams(
            dimension_semantics=("parallel", "arbitrary")),
    )(x, w1b, w2b, b1, b2, g, bb, c0)
    return out.reshape(B, S, D)


# speedup vs baseline: 1.0084x; 1.0084x over previous
import functools

import jax
import jax.numpy as jnp
from jax.experimental import pallas as pl
from jax.experimental.pallas import tpu as pltpu

_TB = 512
_FB = 2048
_SPLIT = 1


def _fused_ffn_kernel(x_ref, w1_ref, w2_ref, b1_ref, b2_ref, g_ref, bb_ref,
                      c0_ref, out_ref, normed_ref, s_ref, *, nj):
    j = pl.program_id(1)

    @pl.when(j == 0)
    def _init():
        xb = x_ref[...]
        mu = jnp.mean(xb, axis=1, keepdims=True)
        xc = xb - mu
        var = jnp.mean(jnp.square(xc), axis=1, keepdims=True)
        normed = xc * jax.lax.rsqrt(var + 1e-5)
        normed_ref[...] = (normed * g_ref[...] + bb_ref[...]).astype(
            jnp.bfloat16)
        s_ref[...] = jax.nn.sigmoid(
            jnp.sum(xb * c0_ref[...], axis=1, keepdims=True))
        out_ref[...] = jnp.zeros_like(out_ref)

    fbk = w1_ref.shape[0] // _SPLIT
    normed = normed_ref[...]
    acc = None
    for k in range(_SPLIT):
        h = jax.lax.dot_general(
            normed, w1_ref[k * fbk:(k + 1) * fbk, :],
            (((1,), (1,)), ((), ())), preferred_element_type=jnp.float32)
        h = jnp.maximum(h + b1_ref[:, k * fbk:(k + 1) * fbk],
                        0.0).astype(jnp.bfloat16)
        d = jax.lax.dot_general(
            h, w2_ref[:, k * fbk:(k + 1) * fbk],
            (((1,), (1,)), ((), ())), preferred_element_type=jnp.float32)
        acc = d if acc is None else acc + d

    out_ref[...] += acc

    @pl.when(j == nj - 1)
    def _finish():
        s = s_ref[...]
        out_ref[...] = x_ref[...] + s * (out_ref[...] + b2_ref[...])


def kernel(input_features, expert_centroids, ln_g, ln_b, ff1_w, ff1_b,
           ff2_w, ff2_b):
    B, S, D = input_features.shape
    FF = ff1_w.shape[0]
    N = B * S
    x = input_features.reshape(N, D)
    w1b = ff1_w.astype(jnp.bfloat16)
    w2b = ff2_w.astype(jnp.bfloat16)
    b1 = ff1_b.reshape(1, FF)
    b2 = ff2_b.reshape(1, D)
    g = ln_g.reshape(1, D)
    bb = ln_b.reshape(1, D)
    c0 = expert_centroids[0].reshape(1, D)
    ni, nj = N // _TB, FF // _FB

    out = pl.pallas_call(
        functools.partial(_fused_ffn_kernel, nj=nj),
        grid=(ni, nj),
        in_specs=[
            pl.BlockSpec((_TB, D), lambda i, j: (i, 0)),
            pl.BlockSpec((_FB, D), lambda i, j: (j, 0)),
            pl.BlockSpec((D, _FB), lambda i, j: (0, j)),
            pl.BlockSpec((1, _FB), lambda i, j: (0, j)),
            pl.BlockSpec((1, D), lambda i, j: (0, 0)),
            pl.BlockSpec((1, D), lambda i, j: (0, 0)),
            pl.BlockSpec((1, D), lambda i, j: (0, 0)),
            pl.BlockSpec((1, D), lambda i, j: (0, 0)),
        ],
        out_specs=pl.BlockSpec((_TB, D), lambda i, j: (i, 0)),
        out_shape=jax.ShapeDtypeStruct((N, D), jnp.float32),
        scratch_shapes=[
            pltpu.VMEM((_TB, D), jnp.bfloat16),
            pltpu.VMEM((_TB, 1), jnp.float32),
        ],
        compiler_params=pltpu.CompilerParams(
            dimension_semantics=("parallel", "arbitrary")),
    )(x, w1b, w2b, b1, b2, g, bb, c0)
    return out.reshape(B, S, D)
